# baseline (speedup 1.0000x reference)
import functools

import jax
import jax.numpy as jnp
from jax import lax
from jax.experimental import pallas as pl
from jax.experimental.pallas import tpu as pltpu
from jax.experimental.pallas import tpu_sc as plsc

NC = 2
NS = 16
NW = NC * NS

D = 64
DP = 128
CH = 128
NBUF = 4
L = 16


@functools.partial(jax.jit, static_argnames=("n_h", "n_b"))
def _gather(idx3, table_p, *, n_h, n_b):
    n_rows = n_h * n_b
    per_w = n_rows // NW
    n_ch = per_w // CH
    mesh = plsc.VectorSubcoreMesh(core_axis_name="c", subcore_axis_name="s")

    @functools.partial(
        pl.kernel,
        out_type=jax.ShapeDtypeStruct((n_h, D, n_b), jnp.float32),
        mesh=mesh,
        scratch_types=[
            pltpu.VMEM((n_ch, CH), jnp.int32),
            [pltpu.VMEM((CH, DP), jnp.float32) for _ in range(NBUF)],
            [pltpu.VMEM((D, CH), jnp.float32) for _ in range(NBUF)],
            [pltpu.SemaphoreType.DMA for _ in range(NBUF)],
            [pltpu.SemaphoreType.DMA for _ in range(NBUF)],
        ],
        compiler_params=pltpu.CompilerParams(needs_layout_passes=False),
    )
    def k(table_hbm, idx_hbm, out_hbm, idx_v, gbufs, tbufs, gsems, wsems):
        wid = lax.axis_index("s") * NC + lax.axis_index("c")
        w_base = wid * per_w

        pltpu.sync_copy(idx_hbm.at[wid], idx_v)

        def gather_chunk(g, b):
            return pltpu.make_async_copy(
                table_hbm.at[idx_v.at[g]], gbufs[b], gsems[b])

        def write_chunk(g, b):
            row = w_base + g * CH
            return pltpu.make_async_copy(
                tbufs[b],
                out_hbm.at[row // n_b, :, pl.ds(row % n_b, CH)],
                wsems[b])

        def transpose_chunk(b):
            gb, tb = gbufs[b], tbufs[b]
            lane = lax.iota(jnp.int32, L)

            def body(d, carry):
                col = jnp.full((L,), d, dtype=jnp.int32)
                for j in range(CH // L):
                    rows = lane + (L * j)
                    vals = plsc.load_gather(gb, [rows, col])
                    tb[d, pl.ds(L * j, L)] = vals
                return carry

            lax.fori_loop(0, D, body, 0)

        for b in range(NBUF - 1):
            gather_chunk(b, b).start()

        def outer(i, carry):
            so = i * NBUF
            for b in range(NBUF):
                g = so + b
                @pl.when(g >= NBUF)
                def _():
                    write_chunk(g - NBUF, b).wait()

                gather_chunk(g, b).wait()
                bn = (b + NBUF - 1) % NBUF

                @pl.when(g + NBUF - 1 < n_ch)
                def _():
                    gather_chunk(g + NBUF - 1, bn).start()

                transpose_chunk(b)
                write_chunk(g, b).start()
            return carry

        lax.fori_loop(0, n_ch // NBUF, outer, 0)

        for j in range(NBUF):
            g = n_ch - NBUF + j
            write_chunk(g, g % NBUF).wait()

    return k(table_p, idx3)


def kernel(imputs, table):
    b, h = imputs.shape
    n_rows = b * h
    per_w = n_rows // NW
    idx3 = imputs.T.reshape(NW, per_w // CH, CH).astype(jnp.int32)
    table_p = jnp.pad(table, ((0, 0), (0, DP - D)))
    out = _gather(idx3, table_p, n_h=h, n_b=b)
    return out.transpose(2, 0, 1)


# speedup vs baseline: 1.4821x; 1.4821x over previous
import functools

import jax
import jax.numpy as jnp
from jax import lax
from jax.experimental import pallas as pl
from jax.experimental.pallas import tpu as pltpu
from jax.experimental.pallas import tpu_sc as plsc

NC = 2
NS = 16
NW = NC * NS

D = 64
DP = 128
CH = 128
NBUF = 4
L = 16


@functools.partial(jax.jit, static_argnames=("n_h", "n_b"))
def _gather(idx3, table_p, *, n_h, n_b):
    n_rows = n_h * n_b
    per_w = n_rows // NW
    n_ch = per_w // CH
    mesh = plsc.VectorSubcoreMesh(core_axis_name="c", subcore_axis_name="s")

    @functools.partial(
        pl.kernel,
        out_type=jax.ShapeDtypeStruct((n_h, D, n_b), jnp.float32),
        mesh=mesh,
        scratch_types=[
            pltpu.VMEM((n_ch, CH), jnp.int32),
            [pltpu.VMEM((CH, DP), jnp.float32) for _ in range(NBUF)],
            [pltpu.VMEM((D, CH), jnp.float32) for _ in range(NBUF)],
            [pltpu.SemaphoreType.DMA for _ in range(NBUF)],
            [pltpu.SemaphoreType.DMA for _ in range(NBUF)],
        ],
        compiler_params=pltpu.CompilerParams(needs_layout_passes=False),
    )
    def k(table_hbm, idx_hbm, out_hbm, idx_v, gbufs, tbufs, gsems, wsems):
        wid = lax.axis_index("s") * NC + lax.axis_index("c")
        w_base = wid * per_w

        pltpu.sync_copy(idx_hbm.at[wid], idx_v)

        def gather_chunk(g, b):
            return pltpu.make_async_copy(
                table_hbm.at[idx_v.at[g]], gbufs[b], gsems[b])

        def write_chunk(g, b):
            row = w_base + g * CH
            return pltpu.make_async_copy(
                tbufs[b],
                out_hbm.at[row // n_b, :, pl.ds(row % n_b, CH)],
                wsems[b])

        def transpose_chunk(b):
            gb, tb = gbufs[b], tbufs[b]
            lane = lax.iota(jnp.int32, L)
            rows = [lane + (L * j) for j in range(CH // L)]

            @plsc.parallel_loop(0, D, step=1, unroll=4)
            def _(d):
                col = jnp.full((L,), d, dtype=jnp.int32)
                for j in range(CH // L):
                    tb[d, pl.ds(L * j, L)] = plsc.load_gather(
                        gb, [rows[j], col])

        for b in range(NBUF - 1):
            gather_chunk(b, b).start()

        def outer(i, carry):
            so = i * NBUF
            for b in range(NBUF):
                g = so + b
                @pl.when(g >= NBUF)
                def _():
                    write_chunk(g - NBUF, b).wait()

                gather_chunk(g, b).wait()
                bn = (b + NBUF - 1) % NBUF

                @pl.when(g + NBUF - 1 < n_ch)
                def _():
                    gather_chunk(g + NBUF - 1, bn).start()

                transpose_chunk(b)
                write_chunk(g, b).start()
            return carry

        lax.fori_loop(0, n_ch // NBUF, outer, 0)

        for j in range(NBUF):
            g = n_ch - NBUF + j
            write_chunk(g, g % NBUF).wait()

    return k(table_p, idx3)


def kernel(imputs, table):
    b, h = imputs.shape
    n_rows = b * h
    per_w = n_rows // NW
    idx3 = imputs.T.reshape(NW, per_w // CH, CH).astype(jnp.int32)
    table_p = jnp.pad(table, ((0, 0), (0, DP - D)))
    out = _gather(idx3, table_p, n_h=h, n_b=b)
    return out.transpose(2, 0, 1)


# speedup vs baseline: 1.4829x; 1.0005x over previous
import functools

import jax
import jax.numpy as jnp
from jax import lax
from jax.experimental import pallas as pl
from jax.experimental.pallas import tpu as pltpu
from jax.experimental.pallas import tpu_sc as plsc

NC = 2
NS = 16
NW = NC * NS

D = 64
DP = 128
CH = 128
NBUF = 4
L = 16


@functools.partial(jax.jit, static_argnames=("n_h", "n_b"))
def _gather(idx3, table_p, *, n_h, n_b):
    n_rows = n_h * n_b
    per_w = n_rows // NW
    n_ch = per_w // CH
    mesh = plsc.VectorSubcoreMesh(core_axis_name="c", subcore_axis_name="s")

    @functools.partial(
        pl.kernel,
        out_type=jax.ShapeDtypeStruct((n_h, D, n_b), jnp.float32),
        mesh=mesh,
        scratch_types=[
            pltpu.VMEM((n_ch, CH), jnp.int32),
            [pltpu.VMEM((CH, DP), jnp.float32) for _ in range(NBUF)],
            [pltpu.VMEM((D, CH), jnp.float32) for _ in range(NBUF)],
            [pltpu.SemaphoreType.DMA for _ in range(NBUF)],
            [pltpu.SemaphoreType.DMA for _ in range(NBUF)],
        ],
        compiler_params=pltpu.CompilerParams(needs_layout_passes=False),
    )
    def k(table_hbm, idx_hbm, out_hbm, idx_v, gbufs, tbufs, gsems, wsems):
        wid = lax.axis_index("s") * NC + lax.axis_index("c")
        w_base = wid * per_w

        pltpu.sync_copy(idx_hbm.at[wid], idx_v)

        def gather_chunk(g, b):
            return pltpu.make_async_copy(
                table_hbm.at[idx_v.at[g]], gbufs[b], gsems[b])

        def write_chunk(g, b):
            row = w_base + g * CH
            return pltpu.make_async_copy(
                tbufs[b],
                out_hbm.at[row // n_b, :, pl.ds(row % n_b, CH)],
                wsems[b])

        def transpose_chunk(b):
            gb, tb = gbufs[b], tbufs[b]
            lane = lax.iota(jnp.int32, L)
            rows = [lane + (L * j) for j in range(CH // L)]

            @plsc.parallel_loop(0, D, step=1, unroll=8)
            def _(d):
                col = jnp.full((L,), d, dtype=jnp.int32)
                for j in range(CH // L):
                    tb[d, pl.ds(L * j, L)] = plsc.load_gather(
                        gb, [rows[j], col])

        for b in range(NBUF - 1):
            gather_chunk(b, b).start()

        def outer(i, carry):
            so = i * NBUF
            for b in range(NBUF):
                g = so + b
                @pl.when(g >= NBUF)
                def _():
                    write_chunk(g - NBUF, b).wait()

                gather_chunk(g, b).wait()
                bn = (b + NBUF - 1) % NBUF

                @pl.when(g + NBUF - 1 < n_ch)
                def _():
                    gather_chunk(g + NBUF - 1, bn).start()

                transpose_chunk(b)
                write_chunk(g, b).start()
            return carry

        lax.fori_loop(0, n_ch // NBUF, outer, 0)

        for j in range(NBUF):
            g = n_ch - NBUF + j
            write_chunk(g, g % NBUF).wait()

    return k(table_p, idx3)


def kernel(imputs, table):
    b, h = imputs.shape
    n_rows = b * h
    per_w = n_rows // NW
    idx3 = imputs.T.reshape(NW, per_w // CH, CH).astype(jnp.int32)
    table_p = jnp.pad(table, ((0, 0), (0, DP - D)))
    out = _gather(idx3, table_p, n_h=h, n_b=b)
    return out.transpose(2, 0, 1)


# speedup vs baseline: 2.3117x; 1.5589x over previous
import functools

import jax
import jax.numpy as jnp
from jax import lax
from jax.experimental import pallas as pl
from jax.experimental.pallas import tpu as pltpu
from jax.experimental.pallas import tpu_sc as plsc

NC = 2
NS = 16
NW = NC * NS

D = 64
DP = 128
CH = 128
NBUF = 4
L = 16


@functools.partial(jax.jit, static_argnames=("n_h", "n_b"))
def _gather(idx3, table_p, *, n_h, n_b):
    n_rows = n_h * n_b
    per_w = n_rows // NW
    n_ch = per_w // CH
    mesh = plsc.VectorSubcoreMesh(core_axis_name="c", subcore_axis_name="s")

    @functools.partial(
        pl.kernel,
        out_type=jax.ShapeDtypeStruct((n_h, D, n_b), jnp.float32),
        mesh=mesh,
        scratch_types=[
            pltpu.VMEM((n_ch, CH), jnp.int32),
            [pltpu.VMEM((CH, DP), jnp.float32) for _ in range(NBUF)],
            [pltpu.VMEM((D, CH), jnp.float32) for _ in range(NBUF)],
            [pltpu.SemaphoreType.DMA for _ in range(NBUF)],
            [pltpu.SemaphoreType.DMA for _ in range(NBUF)],
        ],
        compiler_params=pltpu.CompilerParams(needs_layout_passes=False),
    )
    def k(table_hbm, idx_hbm, out_hbm, idx_v, gbufs, tbufs, gsems, wsems):
        wid = lax.axis_index("s") * NC + lax.axis_index("c")
        w_base = wid * per_w

        pltpu.sync_copy(idx_hbm.at[wid], idx_v)

        def gather_chunk(g, b):
            return pltpu.make_async_copy(
                table_hbm.at[idx_v.at[g]], gbufs[b], gsems[b])

        def write_chunk(g, b):
            row = w_base + g * CH
            return pltpu.make_async_copy(
                tbufs[b],
                out_hbm.at[row // n_b, :, pl.ds(row % n_b, CH)],
                wsems[b])

        def transpose_chunk(b):
            gb, tb = gbufs[b], tbufs[b]
            lane = lax.iota(jnp.int32, L)

            @plsc.parallel_loop(0, (D // L) * (CH // L), step=1)
            def _(i):
                d0 = (i & ((D // L) - 1)) * L
                rows = lane + (i >> 2) * L
                for s in range(L):
                    colv = ((lane + s) & (L - 1)) + d0
                    val = plsc.load_gather(gb, [rows, colv])
                    plsc.store_scatter(tb, [colv, rows], val)

        for b in range(NBUF - 1):
            gather_chunk(b, b).start()

        def outer(i, carry):
            so = i * NBUF
            for b in range(NBUF):
                g = so + b
                @pl.when(g >= NBUF)
                def _():
                    write_chunk(g - NBUF, b).wait()

                gather_chunk(g, b).wait()
                bn = (b + NBUF - 1) % NBUF

                @pl.when(g + NBUF - 1 < n_ch)
                def _():
                    gather_chunk(g + NBUF - 1, bn).start()

                transpose_chunk(b)
                write_chunk(g, b).start()
            return carry

        lax.fori_loop(0, n_ch // NBUF, outer, 0)

        for j in range(NBUF):
            g = n_ch - NBUF + j
            write_chunk(g, g % NBUF).wait()

    return k(table_p, idx3)


def kernel(imputs, table):
    b, h = imputs.shape
    n_rows = b * h
    per_w = n_rows // NW
    idx3 = imputs.T.reshape(NW, per_w // CH, CH).astype(jnp.int32)
    table_p = jnp.pad(table, ((0, 0), (0, DP - D)))
    out = _gather(idx3, table_p, n_h=h, n_b=b)
    return out.transpose(2, 0, 1)
